# Initial kernel scaffold; baseline (speedup 1.0000x reference)
import functools
import math

import jax
import jax.numpy as jnp
from jax import lax
from jax.experimental import pallas as pl
from jax.experimental.pallas import tpu as pltpu
from jax.experimental.pallas import tpu_sc as plsc

N = 10000
E = 160000
D = 128
DE = 4

_NC = 2
_NS = 16
_NW = _NC * _NS
_PW = E // _NW
_CH = 40
_NCH = _PW // _CH

_RPT = N // _NS
_ZB = 125
_NZB = _RPT // _ZB

_MESH = plsc.VectorSubcoreMesh(core_axis_name="c", subcore_axis_name="s")


@functools.partial(
    pl.kernel,
    out_type=jax.ShapeDtypeStruct((E, D), jnp.float32),
    mesh=_MESH,
    scratch_types=[
        pltpu.VMEM((_CH,), jnp.int32),
        pltpu.VMEM((_CH, D), jnp.float32),
        pltpu.SemaphoreType.DMA,
    ],
)
def _sc_gather(x_hbm, src_hbm, xs_hbm, idx_v, rows_v, sem):
    wid = lax.axis_index("s") * _NC + lax.axis_index("c")
    base = wid * _PW

    def body(g, carry):
        off = pl.multiple_of(base + g * _CH, 8)
        pltpu.sync_copy(src_hbm.at[pl.ds(off, _CH)], idx_v)
        pltpu.async_copy(x_hbm.at[idx_v], rows_v, sem).wait()
        pltpu.sync_copy(rows_v, xs_hbm.at[pl.ds(off, _CH)])
        return carry

    lax.fori_loop(0, _NCH, body, 0)


@functools.partial(
    pl.kernel,
    out_type=jax.ShapeDtypeStruct((_NC, N, D), jnp.float32),
    mesh=_MESH,
    scratch_types=[
        pltpu.VMEM((_CH,), jnp.int32),
        pltpu.VMEM((_CH, D), jnp.float32),
        pltpu.VMEM((_ZB, D), jnp.float32),
        pltpu.VMEM_SHARED((N, D), jnp.float32),
        pltpu.SemaphoreType.DMA,
    ],
)
def _sc_scatter(msg_hbm, dst_hbm, agg_hbm, idx_v, rows_v, znc_v, acc_sh, sem):
    c = lax.axis_index("c")
    s = lax.axis_index("s")
    wid = s * _NC + c

    zvec = jnp.zeros((16,), jnp.float32)

    def zbody(i, carry):
        for cc in range(D // 16):
            znc_v[i, pl.ds(cc * 16, 16)] = zvec
        return carry

    lax.fori_loop(0, _ZB, zbody, 0)
    for b in range(_NZB):
        r0 = s * _RPT + b * _ZB
        pltpu.sync_copy(znc_v, acc_sh.at[pl.ds(r0, _ZB)])
    plsc.subcore_barrier()

    def body(g, carry):
        off = pl.multiple_of(wid * _PW + g * _CH, 8)
        pltpu.sync_copy(dst_hbm.at[pl.ds(off, _CH)], idx_v)
        pltpu.sync_copy(msg_hbm.at[pl.ds(off, _CH)], rows_v)
        pltpu.sync_copy(rows_v, acc_sh.at[idx_v], add=True)
        return carry

    lax.fori_loop(0, _NCH, body, 0)
    plsc.subcore_barrier()

    for b in range(_NZB):
        r0 = s * _RPT + b * _ZB
        pltpu.sync_copy(acc_sh.at[pl.ds(r0, _ZB)], znc_v)
        pltpu.sync_copy(znc_v, agg_hbm.at[c, pl.ds(r0, _ZB)])


_BE = 1000
_INV_NORM = 1.0 / math.sqrt(float(D * DE))
_DN = (((1,), (0,)), ((), ()))


def _msg_body(xs_ref, ef_ref, es_ref, wtp_ref, wr_ref, br_ref, out_ref):
    xs = xs_ref[...]
    ef = ef_ref[...]
    es = es_ref[...]
    scale = jax.nn.sigmoid(
        lax.dot_general(es, wr_ref[...], _DN, preferred_element_type=jnp.float32)
        + br_ref[...]
    )
    acc = lax.dot_general(xs, wtp_ref[0], _DN, preferred_element_type=jnp.float32) * ef[:, 0:1]
    for j in range(1, DE):
        acc += lax.dot_general(xs, wtp_ref[j], _DN, preferred_element_type=jnp.float32) * ef[:, j : j + 1]
    out_ref[...] = acc * scale * _INV_NORM


_tc_messages = pl.pallas_call(
    _msg_body,
    grid=(E // _BE,),
    in_specs=[
        pl.BlockSpec((_BE, D), lambda i: (i, 0)),
        pl.BlockSpec((_BE, DE), lambda i: (i, 0)),
        pl.BlockSpec((_BE, DE), lambda i: (i, 0)),
        pl.BlockSpec((DE, D, D), lambda i: (0, 0, 0)),
        pl.BlockSpec((DE, D), lambda i: (0, 0)),
        pl.BlockSpec((1, D), lambda i: (0, 0)),
    ],
    out_specs=pl.BlockSpec((_BE, D), lambda i: (i, 0)),
    out_shape=jax.ShapeDtypeStruct((E, D), jnp.float32),
)


_BN = 1000


def _fin_body(x_ref, a0_ref, a1_ref, wl_ref, out_ref):
    xa = x_ref[...] + a0_ref[...] + a1_ref[...]
    out_ref[...] = lax.dot_general(xa, wl_ref[...], _DN, preferred_element_type=jnp.float32)


_tc_final = pl.pallas_call(
    _fin_body,
    grid=(N // _BN,),
    in_specs=[
        pl.BlockSpec((_BN, D), lambda i: (i, 0)),
        pl.BlockSpec((_BN, D), lambda i: (i, 0)),
        pl.BlockSpec((_BN, D), lambda i: (i, 0)),
        pl.BlockSpec((D, D), lambda i: (0, 0)),
    ],
    out_specs=pl.BlockSpec((_BN, D), lambda i: (i, 0)),
    out_shape=jax.ShapeDtypeStruct((N, D), jnp.float32),
)


def kernel(x, edge_index, edge_feat, edge_scalars, W_tp, W_radial, b_radial, W_lin):
    src = edge_index[0]
    dst = edge_index[1]
    xs = _sc_gather(x, src)
    wtp_t = jnp.transpose(W_tp, (1, 0, 2))
    msg = _tc_messages(xs, edge_feat, edge_scalars, wtp_t, W_radial,
                       b_radial.reshape(1, D))
    agg = _sc_scatter(msg, dst)
    out = _tc_final(x, agg[0], agg[1], W_lin / math.sqrt(float(D)))
    return out


# speedup vs baseline: 1.7236x; 1.7236x over previous
import functools
import math

import jax
import jax.numpy as jnp
from jax import lax
from jax.experimental import pallas as pl
from jax.experimental.pallas import tpu as pltpu
from jax.experimental.pallas import tpu_sc as plsc

N = 10000
E = 160000
D = 128
DE = 4

_NC = 2
_NS = 16
_NW = _NC * _NS
_PW = E // _NW
_CH = 40
_NCH = _PW // _CH

_NP = 10240
_RPT = _NP // _NS
_ZB = 128
_NZB = _RPT // _ZB

_MESH = plsc.VectorSubcoreMesh(core_axis_name="c", subcore_axis_name="s")


@functools.partial(
    pl.kernel,
    out_type=jax.ShapeDtypeStruct((E, D), jnp.float32),
    mesh=_MESH,
    scratch_types=[
        pltpu.VMEM((_CH,), jnp.int32),
        pltpu.VMEM((_CH, D), jnp.float32),
        pltpu.SemaphoreType.DMA,
    ],
)
def _sc_gather(x_hbm, src_hbm, xs_hbm, idx_v, rows_v, sem):
    wid = lax.axis_index("s") * _NC + lax.axis_index("c")
    base = wid * _PW

    def body(g, carry):
        off = pl.multiple_of(base + g * _CH, 8)
        pltpu.sync_copy(src_hbm.at[pl.ds(off, _CH)], idx_v)
        pltpu.async_copy(x_hbm.at[idx_v], rows_v, sem).wait()
        pltpu.sync_copy(rows_v, xs_hbm.at[pl.ds(off, _CH)])
        return carry

    lax.fori_loop(0, _NCH, body, 0)


@functools.partial(
    pl.kernel,
    out_type=jax.ShapeDtypeStruct((_NC, _NP, D), jnp.float32),
    mesh=_MESH,
    scratch_types=[
        pltpu.VMEM((_CH,), jnp.int32),
        pltpu.VMEM((_CH, D), jnp.float32),
        pltpu.VMEM((_ZB, D), jnp.float32),
        pltpu.VMEM_SHARED((_NP, D), jnp.float32),
        pltpu.SemaphoreType.DMA,
    ],
)
def _sc_scatter(msg_hbm, dst_hbm, agg_hbm, idx_v, rows_v, znc_v, acc_sh, sem):
    c = lax.axis_index("c")
    s = lax.axis_index("s")
    wid = s * _NC + c

    zvec = jnp.zeros((16,), jnp.float32)

    def zbody(i, carry):
        for cc in range(D // 16):
            znc_v[i, pl.ds(cc * 16, 16)] = zvec
        return carry

    lax.fori_loop(0, _ZB, zbody, 0)
    for b in range(_NZB):
        r0 = s * _RPT + b * _ZB
        pltpu.sync_copy(znc_v, acc_sh.at[pl.ds(r0, _ZB)])
    plsc.subcore_barrier()

    def body(g, carry):
        off = pl.multiple_of(wid * _PW + g * _CH, 8)
        pltpu.sync_copy(dst_hbm.at[pl.ds(off, _CH)], idx_v)
        pltpu.sync_copy(msg_hbm.at[pl.ds(off, _CH)], rows_v)
        pltpu.sync_copy(rows_v, acc_sh.at[idx_v], add=True)
        return carry

    lax.fori_loop(0, _NCH, body, 0)
    plsc.subcore_barrier()

    for b in range(_NZB):
        r0 = s * _RPT + b * _ZB
        pltpu.sync_copy(acc_sh.at[pl.ds(r0, _ZB)], znc_v)
        pltpu.sync_copy(znc_v, agg_hbm.at[c, pl.ds(r0, _ZB)])


_BE = 1000
_INV_NORM = 1.0 / math.sqrt(float(D * DE))
_DN = (((1,), (0,)), ((), ()))


def _msg_body(xs_ref, ef_ref, es_ref, wtp_ref, wr_ref, br_ref, out_ref):
    xs = xs_ref[...]
    ef = ef_ref[...]
    es = es_ref[...]
    scale = jax.nn.sigmoid(
        lax.dot_general(es, wr_ref[...], _DN, preferred_element_type=jnp.float32)
        + br_ref[...]
    )
    acc = lax.dot_general(xs, wtp_ref[0], _DN, preferred_element_type=jnp.float32) * ef[:, 0:1]
    for j in range(1, DE):
        acc += lax.dot_general(xs, wtp_ref[j], _DN, preferred_element_type=jnp.float32) * ef[:, j : j + 1]
    out_ref[...] = acc * scale * _INV_NORM


_tc_messages = pl.pallas_call(
    _msg_body,
    grid=(E // _BE,),
    in_specs=[
        pl.BlockSpec((_BE, D), lambda i: (i, 0)),
        pl.BlockSpec((_BE, DE), lambda i: (i, 0)),
        pl.BlockSpec((_BE, DE), lambda i: (i, 0)),
        pl.BlockSpec((DE, D, D), lambda i: (0, 0, 0)),
        pl.BlockSpec((DE, D), lambda i: (0, 0)),
        pl.BlockSpec((1, D), lambda i: (0, 0)),
    ],
    out_specs=pl.BlockSpec((_BE, D), lambda i: (i, 0)),
    out_shape=jax.ShapeDtypeStruct((E, D), jnp.float32),
)


_BN = 1000


def _fin_body(x_ref, a0_ref, a1_ref, wl_ref, out_ref):
    xa = x_ref[...] + a0_ref[...] + a1_ref[...]
    out_ref[...] = lax.dot_general(xa, wl_ref[...], _DN, preferred_element_type=jnp.float32)


_tc_final = pl.pallas_call(
    _fin_body,
    grid=(N // _BN,),
    in_specs=[
        pl.BlockSpec((_BN, D), lambda i: (i, 0)),
        pl.BlockSpec((_BN, D), lambda i: (i, 0)),
        pl.BlockSpec((_BN, D), lambda i: (i, 0)),
        pl.BlockSpec((D, D), lambda i: (0, 0)),
    ],
    out_specs=pl.BlockSpec((_BN, D), lambda i: (i, 0)),
    out_shape=jax.ShapeDtypeStruct((N, D), jnp.float32),
)


def kernel(x, edge_index, edge_feat, edge_scalars, W_tp, W_radial, b_radial, W_lin):
    src = edge_index[0]
    dst = edge_index[1]
    xs = _sc_gather(x, src)
    wtp_t = jnp.transpose(W_tp, (1, 0, 2))
    msg = _tc_messages(xs, edge_feat, edge_scalars, wtp_t, W_radial,
                       b_radial.reshape(1, D))
    agg = _sc_scatter(msg, dst)
    out = _tc_final(x, agg[0, :N], agg[1, :N], W_lin / math.sqrt(float(D)))
    return out


# speedup vs baseline: 2.5980x; 1.5073x over previous
import functools
import math

import jax
import jax.numpy as jnp
from jax import lax
from jax.experimental import pallas as pl
from jax.experimental.pallas import tpu as pltpu
from jax.experimental.pallas import tpu_sc as plsc

N = 10000
E = 160000
D = 128
DE = 4

_NC = 2
_NS = 16
_NW = _NC * _NS
_PW = E // _NW
_CH = 40
_NCH = _PW // _CH
_NB = 5
_NG = _NCH // _NB

_NP = 10240
_RPT = _NP // _NS
_ZB = 32
_NZB = _RPT // _ZB

_MESH = plsc.VectorSubcoreMesh(core_axis_name="c", subcore_axis_name="s")


@functools.partial(
    pl.kernel,
    out_type=jax.ShapeDtypeStruct((E, D), jnp.float32),
    mesh=_MESH,
    scratch_types=[
        pltpu.VMEM((_NCH, _CH), jnp.int32),
        [pltpu.VMEM((_CH, D), jnp.float32) for _ in range(_NB)],
        pltpu.SemaphoreType.DMA((_NB,)),
        pltpu.SemaphoreType.DMA((_NB,)),
    ],
)
def _sc_gather(x_hbm, src_hbm, xs_hbm, idx_v, rows_v, gsem, ssem):
    wid = lax.axis_index("s") * _NC + lax.axis_index("c")
    base = wid * _PW
    pltpu.sync_copy(src_hbm.at[wid], idx_v)

    def _gather(g, b):
        pltpu.async_copy(x_hbm.at[idx_v.at[g]], rows_v[b], gsem.at[b])

    def _store(g, b):
        off = pl.multiple_of(base + g * _CH, 8)
        pltpu.async_copy(rows_v[b], xs_hbm.at[pl.ds(off, _CH)], ssem.at[b])

    def _wait_gather(b):
        pltpu.make_async_copy(x_hbm.at[idx_v.at[0]], rows_v[b], gsem.at[b]).wait()

    def _wait_store(b):
        pltpu.make_async_copy(rows_v[b], xs_hbm.at[pl.ds(base, _CH)], ssem.at[b]).wait()

    for b in range(_NB):
        _gather(b, b)

    def body(k, carry):
        t = k * _NB
        for b in range(_NB):
            _wait_gather(b)
            _store(t + b, b)
        for b in range(_NB):
            _wait_store(b)

            @pl.when(k < _NG - 1)
            def _():
                _gather(t + _NB + b, b)
        return carry

    lax.fori_loop(0, _NG, body, 0)


@functools.partial(
    pl.kernel,
    out_type=jax.ShapeDtypeStruct((_NC, _NP, D), jnp.float32),
    mesh=_MESH,
    scratch_types=[
        pltpu.VMEM((_NCH, _CH), jnp.int32),
        [pltpu.VMEM((_CH, D), jnp.float32) for _ in range(_NB)],
        pltpu.VMEM((_ZB, D), jnp.float32),
        pltpu.VMEM_SHARED((_NP, D), jnp.float32),
        pltpu.SemaphoreType.DMA((_NB,)),
        pltpu.SemaphoreType.DMA((_NB,)),
    ],
)
def _sc_scatter(msg_hbm, dst_hbm, agg_hbm, idx_v, rows_v, znc_v, acc_sh,
                lsem, asem):
    c = lax.axis_index("c")
    s = lax.axis_index("s")
    wid = s * _NC + c
    base = wid * _PW

    zvec = jnp.zeros((16,), jnp.float32)

    def zbody(i, carry):
        for cc in range(D // 16):
            znc_v[i, pl.ds(cc * 16, 16)] = zvec
        return carry

    lax.fori_loop(0, _ZB, zbody, 0)
    for b in range(_NZB):
        r0 = s * _RPT + b * _ZB
        pltpu.sync_copy(znc_v, acc_sh.at[pl.ds(r0, _ZB)])
    plsc.subcore_barrier()

    pltpu.sync_copy(dst_hbm.at[wid], idx_v)

    def _load(g, b):
        off = pl.multiple_of(base + g * _CH, 8)
        pltpu.async_copy(msg_hbm.at[pl.ds(off, _CH)], rows_v[b], lsem.at[b])

    def _scat(g, b):
        pltpu.async_copy(rows_v[b], acc_sh.at[idx_v.at[g]], asem.at[b], add=True)

    def _wait_load(b):
        pltpu.make_async_copy(msg_hbm.at[pl.ds(base, _CH)], rows_v[b], lsem.at[b]).wait()

    def _wait_scat(b):
        pltpu.make_async_copy(rows_v[b], acc_sh.at[idx_v.at[0]], asem.at[b]).wait()

    for b in range(_NB):
        _load(b, b)

    def body(k, carry):
        t = k * _NB
        for b in range(_NB):
            _wait_load(b)
            _scat(t + b, b)
        for b in range(_NB):
            _wait_scat(b)

            @pl.when(k < _NG - 1)
            def _():
                _load(t + _NB + b, b)
        return carry

    lax.fori_loop(0, _NG, body, 0)
    plsc.subcore_barrier()

    for b in range(_NZB):
        r0 = s * _RPT + b * _ZB
        pltpu.sync_copy(acc_sh.at[pl.ds(r0, _ZB)], znc_v)
        pltpu.sync_copy(znc_v, agg_hbm.at[c, pl.ds(r0, _ZB)])


_BE = 1000
_INV_NORM = 1.0 / math.sqrt(float(D * DE))
_DN = (((1,), (0,)), ((), ()))


def _msg_body(xs_ref, ef_ref, es_ref, wtp_ref, wr_ref, br_ref, out_ref):
    xs = xs_ref[...]
    ef = ef_ref[...]
    es = es_ref[...]
    scale = jax.nn.sigmoid(
        lax.dot_general(es, wr_ref[...], _DN, preferred_element_type=jnp.float32)
        + br_ref[...]
    )
    outer = jnp.concatenate(
        [(xs * ef[:, j : j + 1]).astype(jnp.bfloat16) for j in range(DE)], axis=1
    )
    acc = lax.dot_general(outer, wtp_ref[...], _DN, preferred_element_type=jnp.float32)
    out_ref[...] = acc * scale * _INV_NORM


_tc_messages = pl.pallas_call(
    _msg_body,
    grid=(E // _BE,),
    in_specs=[
        pl.BlockSpec((_BE, D), lambda i: (i, 0)),
        pl.BlockSpec((_BE, DE), lambda i: (i, 0)),
        pl.BlockSpec((_BE, DE), lambda i: (i, 0)),
        pl.BlockSpec((DE * D, D), lambda i: (0, 0)),
        pl.BlockSpec((DE, D), lambda i: (0, 0)),
        pl.BlockSpec((1, D), lambda i: (0, 0)),
    ],
    out_specs=pl.BlockSpec((_BE, D), lambda i: (i, 0)),
    out_shape=jax.ShapeDtypeStruct((E, D), jnp.float32),
)


_BN = 1000


def _fin_body(x_ref, a0_ref, a1_ref, wl_ref, out_ref):
    xa = x_ref[...] + a0_ref[...] + a1_ref[...]
    out_ref[...] = lax.dot_general(xa, wl_ref[...], _DN, preferred_element_type=jnp.float32)


_tc_final = pl.pallas_call(
    _fin_body,
    grid=(N // _BN,),
    in_specs=[
        pl.BlockSpec((_BN, D), lambda i: (i, 0)),
        pl.BlockSpec((_BN, D), lambda i: (i, 0)),
        pl.BlockSpec((_BN, D), lambda i: (i, 0)),
        pl.BlockSpec((D, D), lambda i: (0, 0)),
    ],
    out_specs=pl.BlockSpec((_BN, D), lambda i: (i, 0)),
    out_shape=jax.ShapeDtypeStruct((N, D), jnp.float32),
)


def kernel(x, edge_index, edge_feat, edge_scalars, W_tp, W_radial, b_radial, W_lin):
    src = edge_index[0].reshape(_NW, _NCH, _CH)
    dst = edge_index[1].reshape(_NW, _NCH, _CH)
    xs = _sc_gather(x, src)
    wtp_flat = jnp.transpose(W_tp, (1, 0, 2)).reshape(DE * D, D).astype(jnp.bfloat16)
    msg = _tc_messages(xs, edge_feat, edge_scalars, wtp_flat, W_radial,
                       b_radial.reshape(1, D))
    agg = _sc_scatter(msg, dst)
    out = _tc_final(x, agg[0, :N], agg[1, :N], W_lin / math.sqrt(float(D)))
    return out


# speedup vs baseline: 3.6665x; 1.4113x over previous
import functools
import math

import jax
import jax.numpy as jnp
from jax import lax
from jax.experimental import pallas as pl
from jax.experimental.pallas import tpu as pltpu
from jax.experimental.pallas import tpu_sc as plsc

N = 10000
E = 160000
D = 128
DE = 4

_E1 = 83200
_E2 = 76800

_NC = 2
_NS = 16
_NW = _NC * _NS
_CH = 40
_NB = 5
_NCH1 = _E1 // _NW // _CH
_NCH2 = _E2 // _NW // _CH

_NP = 10240
_RPT = _NP // _NS
_ZB = 32
_NZB = _RPT // _ZB

_MESH = plsc.VectorSubcoreMesh(core_axis_name="c", subcore_axis_name="s")


def _make_gather(ecount, nch):
    ng = nch // _NB
    pw = ecount // _NW

    @functools.partial(
        pl.kernel,
        out_type=jax.ShapeDtypeStruct((ecount, D), jnp.float32),
        mesh=_MESH,
        scratch_types=[
            pltpu.VMEM((nch, _CH), jnp.int32),
            [pltpu.VMEM((_CH, D), jnp.float32) for _ in range(_NB)],
            pltpu.SemaphoreType.DMA((_NB,)),
            pltpu.SemaphoreType.DMA((_NB,)),
        ],
    )
    def _g(x_hbm, src_hbm, xs_hbm, idx_v, rows_v, gsem, ssem):
        wid = lax.axis_index("s") * _NC + lax.axis_index("c")
        base = wid * pw
        pltpu.sync_copy(src_hbm.at[wid], idx_v)

        def _gather(g, b):
            pltpu.async_copy(x_hbm.at[idx_v.at[g]], rows_v[b], gsem.at[b])

        def _store(g, b):
            off = pl.multiple_of(base + g * _CH, 8)
            pltpu.async_copy(rows_v[b], xs_hbm.at[pl.ds(off, _CH)], ssem.at[b])

        def _wait_gather(b):
            pltpu.make_async_copy(x_hbm.at[idx_v.at[0]], rows_v[b], gsem.at[b]).wait()

        def _wait_store(b):
            pltpu.make_async_copy(rows_v[b], xs_hbm.at[pl.ds(base, _CH)], ssem.at[b]).wait()

        for b in range(_NB):
            _gather(b, b)

        def body(k, carry):
            t = k * _NB
            for b in range(_NB):
                _wait_gather(b)
                _store(t + b, b)
            for b in range(_NB):
                _wait_store(b)

                @pl.when(k < ng - 1)
                def _():
                    _gather(t + _NB + b, b)
            return carry

        lax.fori_loop(0, ng, body, 0)

    return _g


_sc_gather1 = _make_gather(_E1, _NCH1)
_sc_gather2 = _make_gather(_E2, _NCH2)


@functools.partial(
    pl.kernel,
    out_type=jax.ShapeDtypeStruct((_NC, _NP, D), jnp.float32),
    mesh=_MESH,
    scratch_types=[
        pltpu.VMEM((_NCH1, _CH), jnp.int32),
        pltpu.VMEM((_NCH2, _CH), jnp.int32),
        [pltpu.VMEM((_CH, D), jnp.float32) for _ in range(_NB)],
        pltpu.VMEM((_ZB, D), jnp.float32),
        pltpu.VMEM_SHARED((_NP, D), jnp.float32),
        pltpu.SemaphoreType.DMA((_NB,)),
        pltpu.SemaphoreType.DMA((_NB,)),
    ],
)
def _sc_scatter(msg1_hbm, dst1_hbm, msg2_hbm, dst2_hbm, agg_hbm,
                idx1_v, idx2_v, rows_v, znc_v, acc_sh, lsem, asem):
    c = lax.axis_index("c")
    s = lax.axis_index("s")
    wid = s * _NC + c

    zvec = jnp.zeros((16,), jnp.float32)

    def zbody(i, carry):
        for cc in range(D // 16):
            znc_v[i, pl.ds(cc * 16, 16)] = zvec
        return carry

    lax.fori_loop(0, _ZB, zbody, 0)
    for b in range(_NZB):
        r0 = s * _RPT + b * _ZB
        pltpu.sync_copy(znc_v, acc_sh.at[pl.ds(r0, _ZB)])
    plsc.subcore_barrier()

    def _run(msg_hbm, idx_v, nch):
        ng = nch // _NB
        pw = nch * _CH
        base = wid * pw

        def _load(g, b):
            off = pl.multiple_of(base + g * _CH, 8)
            pltpu.async_copy(msg_hbm.at[pl.ds(off, _CH)], rows_v[b], lsem.at[b])

        def _scat(g, b):
            pltpu.async_copy(rows_v[b], acc_sh.at[idx_v.at[g]], asem.at[b], add=True)

        def _wait_load(b):
            pltpu.make_async_copy(msg_hbm.at[pl.ds(base, _CH)], rows_v[b], lsem.at[b]).wait()

        def _wait_scat(b):
            pltpu.make_async_copy(rows_v[b], acc_sh.at[idx_v.at[0]], asem.at[b]).wait()

        for b in range(_NB):
            _load(b, b)

        def body(k, carry):
            t = k * _NB
            for b in range(_NB):
                _wait_load(b)
                _scat(t + b, b)
            for b in range(_NB):
                _wait_scat(b)

                @pl.when(k < ng - 1)
                def _():
                    _load(t + _NB + b, b)
            return carry

        lax.fori_loop(0, ng, body, 0)

    pltpu.sync_copy(dst1_hbm.at[wid], idx1_v)
    _run(msg1_hbm, idx1_v, _NCH1)
    pltpu.sync_copy(dst2_hbm.at[wid], idx2_v)
    _run(msg2_hbm, idx2_v, _NCH2)
    plsc.subcore_barrier()

    for b in range(_NZB):
        r0 = s * _RPT + b * _ZB
        pltpu.sync_copy(acc_sh.at[pl.ds(r0, _ZB)], znc_v)
        pltpu.sync_copy(znc_v, agg_hbm.at[c, pl.ds(r0, _ZB)])


_BE = 1280
_INV_NORM = 1.0 / math.sqrt(float(D * DE))
_DN = (((1,), (0,)), ((), ()))


def _msg_body(xs_ref, fs_ref, wtp_ref, wr_ref, br_ref, out_ref):
    xs = xs_ref[...]
    fst = jnp.transpose(fs_ref[...])
    scale = jax.nn.sigmoid(
        lax.dot_general(fst[:, DE:], wr_ref[...], _DN,
                        preferred_element_type=jnp.float32)
        + br_ref[...]
    )
    y = lax.dot_general(xs.astype(jnp.bfloat16), wtp_ref[...], _DN,
                        preferred_element_type=jnp.float32)
    acc = fst[:, 0:1] * y[:, :D]
    for j in range(1, DE):
        acc += fst[:, j : j + 1] * y[:, j * D : (j + 1) * D]
    out_ref[...] = acc * scale


def _make_messages(ecount):
    return pl.pallas_call(
        _msg_body,
        grid=(ecount // _BE,),
        in_specs=[
            pl.BlockSpec((_BE, D), lambda i: (i, 0)),
            pl.BlockSpec((2 * DE, _BE), lambda i: (0, i)),
            pl.BlockSpec((D, DE * D), lambda i: (0, 0)),
            pl.BlockSpec((DE, D), lambda i: (0, 0)),
            pl.BlockSpec((1, D), lambda i: (0, 0)),
        ],
        out_specs=pl.BlockSpec((_BE, D), lambda i: (i, 0)),
        out_shape=jax.ShapeDtypeStruct((ecount, D), jnp.float32),
    )


_tc_messages1 = _make_messages(_E1)
_tc_messages2 = _make_messages(_E2)


_BN = 1000


def _fin_body(x_ref, a0_ref, a1_ref, wl_ref, out_ref):
    xa = x_ref[...] + a0_ref[0] + a1_ref[0]
    out_ref[...] = lax.dot_general(xa, wl_ref[...], _DN, preferred_element_type=jnp.float32)


_tc_final = pl.pallas_call(
    _fin_body,
    grid=(N // _BN,),
    in_specs=[
        pl.BlockSpec((_BN, D), lambda i: (i, 0)),
        pl.BlockSpec((1, _BN, D), lambda i: (0, i, 0)),
        pl.BlockSpec((1, _BN, D), lambda i: (1, i, 0)),
        pl.BlockSpec((D, D), lambda i: (0, 0)),
    ],
    out_specs=pl.BlockSpec((_BN, D), lambda i: (i, 0)),
    out_shape=jax.ShapeDtypeStruct((N, D), jnp.float32),
)


def kernel(x, edge_index, edge_feat, edge_scalars, W_tp, W_radial, b_radial, W_lin):
    src = edge_index[0]
    dst = edge_index[1]
    src1 = src[:_E1].reshape(_NW, _NCH1, _CH)
    src2 = src[_E1:].reshape(_NW, _NCH2, _CH)
    dst1 = dst[:_E1].reshape(_NW, _NCH1, _CH)
    dst2 = dst[_E1:].reshape(_NW, _NCH2, _CH)
    fs = jnp.concatenate([edge_feat.T, edge_scalars.T], axis=0)
    wtp_flat = (W_tp.reshape(D, DE * D) * _INV_NORM).astype(jnp.bfloat16)
    br = b_radial.reshape(1, D)

    xs1 = _sc_gather1(x, src1)
    msg1 = _tc_messages1(xs1, fs[:, :_E1], wtp_flat, W_radial, br)
    xs2 = _sc_gather2(x, src2)
    msg2 = _tc_messages2(xs2, fs[:, _E1:], wtp_flat, W_radial, br)
    agg = _sc_scatter(msg1, dst1, msg2, dst2)
    out = _tc_final(x, agg, agg, W_lin / math.sqrt(float(D)))
    return out


# speedup vs baseline: 3.8866x; 1.0600x over previous
import functools
import math

import jax
import jax.numpy as jnp
from jax import lax
from jax.experimental import pallas as pl
from jax.experimental.pallas import tpu as pltpu
from jax.experimental.pallas import tpu_sc as plsc

N = 10000
E = 160000
D = 128
DE = 4

_E1 = 83200
_E2 = 76800

_NC = 2
_NS = 16
_NW = _NC * _NS
_CH = 40
_NB = 5
_NCH1 = _E1 // _NW // _CH
_NCH2 = _E2 // _NW // _CH

_NP = 10240
_RPT = _NP // _NS
_ZB = 32
_NZB = _RPT // _ZB

_MESH = plsc.VectorSubcoreMesh(core_axis_name="c", subcore_axis_name="s")


def _make_gather(ecount, nch):
    ng = nch // _NB
    pw = ecount // _NW

    @functools.partial(
        pl.kernel,
        out_type=jax.ShapeDtypeStruct((ecount, D), jnp.float32),
        mesh=_MESH,
        scratch_types=[
            pltpu.VMEM((nch, _CH), jnp.int32),
            [pltpu.VMEM((_CH, D), jnp.float32) for _ in range(_NB)],
            pltpu.SemaphoreType.DMA((_NB,)),
            pltpu.SemaphoreType.DMA((_NB,)),
        ],
    )
    def _g(x_hbm, src_hbm, xs_hbm, idx_v, rows_v, gsem, ssem):
        wid = lax.axis_index("s") * _NC + lax.axis_index("c")
        base = wid * pw
        pltpu.sync_copy(src_hbm.at[wid], idx_v)

        def _gather(g, b):
            pltpu.async_copy(x_hbm.at[idx_v.at[g]], rows_v[b], gsem.at[b])

        def _store(g, b):
            off = pl.multiple_of(base + g * _CH, 8)
            pltpu.async_copy(rows_v[b], xs_hbm.at[pl.ds(off, _CH)], ssem.at[b])

        def _wait_gather(b):
            pltpu.make_async_copy(x_hbm.at[idx_v.at[0]], rows_v[b], gsem.at[b]).wait()

        def _wait_store(b):
            pltpu.make_async_copy(rows_v[b], xs_hbm.at[pl.ds(base, _CH)], ssem.at[b]).wait()

        for b in range(_NB):
            _gather(b, b)

        def body(k, carry):
            t = k * _NB
            for b in range(_NB):
                _wait_gather(b)
                _store(t + b, b)
            for b in range(_NB):
                _wait_store(b)

                @pl.when(k < ng - 1)
                def _():
                    _gather(t + _NB + b, b)
            return carry

        lax.fori_loop(0, ng, body, 0)

    return _g


_sc_gather1 = _make_gather(_E1, _NCH1)
_sc_gather2 = _make_gather(_E2, _NCH2)


def _make_scatter(ecount, nch):
    ng = nch // _NB
    pw = ecount // _NW

    @functools.partial(
        pl.kernel,
        out_type=jax.ShapeDtypeStruct((_NC, _NP, D), jnp.float32),
        mesh=_MESH,
        scratch_types=[
            pltpu.VMEM((nch, _CH), jnp.int32),
            [pltpu.VMEM((_CH, D), jnp.float32) for _ in range(_NB)],
            pltpu.VMEM((_ZB, D), jnp.float32),
            pltpu.VMEM_SHARED((_NP, D), jnp.float32),
            pltpu.SemaphoreType.DMA((_NB,)),
            pltpu.SemaphoreType.DMA((_NB,)),
        ],
    )
    def _sc(msg_hbm, dst_hbm, agg_hbm, idx_v, rows_v, znc_v, acc_sh, lsem, asem):
        c = lax.axis_index("c")
        s = lax.axis_index("s")
        wid = s * _NC + c
        base = wid * pw

        zvec = jnp.zeros((16,), jnp.float32)

        def zbody(i, carry):
            for cc in range(D // 16):
                znc_v[i, pl.ds(cc * 16, 16)] = zvec
            return carry

        lax.fori_loop(0, _ZB, zbody, 0)
        for b in range(_NZB):
            r0 = s * _RPT + b * _ZB
            pltpu.sync_copy(znc_v, acc_sh.at[pl.ds(r0, _ZB)])
        plsc.subcore_barrier()

        pltpu.sync_copy(dst_hbm.at[wid], idx_v)

        def _load(g, b):
            off = pl.multiple_of(base + g * _CH, 8)
            pltpu.async_copy(msg_hbm.at[pl.ds(off, _CH)], rows_v[b], lsem.at[b])

        def _scat(g, b):
            pltpu.async_copy(rows_v[b], acc_sh.at[idx_v.at[g]], asem.at[b], add=True)

        def _wait_load(b):
            pltpu.make_async_copy(msg_hbm.at[pl.ds(base, _CH)], rows_v[b], lsem.at[b]).wait()

        def _wait_scat(b):
            pltpu.make_async_copy(rows_v[b], acc_sh.at[idx_v.at[0]], asem.at[b]).wait()

        for b in range(_NB):
            _load(b, b)

        def body(k, carry):
            t = k * _NB
            for b in range(_NB):
                _wait_load(b)
                _scat(t + b, b)
            for b in range(_NB):
                _wait_scat(b)

                @pl.when(k < ng - 1)
                def _():
                    _load(t + _NB + b, b)
            return carry

        lax.fori_loop(0, ng, body, 0)
        plsc.subcore_barrier()

        for b in range(_NZB):
            r0 = s * _RPT + b * _ZB
            pltpu.sync_copy(acc_sh.at[pl.ds(r0, _ZB)], znc_v)
            pltpu.sync_copy(znc_v, agg_hbm.at[c, pl.ds(r0, _ZB)])

    return _sc


_sc_scatter1 = _make_scatter(_E1, _NCH1)
_sc_scatter2 = _make_scatter(_E2, _NCH2)


_BE = 1280
_INV_NORM = 1.0 / math.sqrt(float(D * DE))
_DN = (((1,), (0,)), ((), ()))


def _msg_body(xs_ref, fs_ref, wtp_ref, wr_ref, br_ref, out_ref):
    xs = xs_ref[...]
    fst = jnp.transpose(fs_ref[...])
    scale = jax.nn.sigmoid(
        lax.dot_general(fst[:, DE:], wr_ref[...], _DN,
                        preferred_element_type=jnp.float32)
        + br_ref[...]
    )
    y = lax.dot_general(xs.astype(jnp.bfloat16), wtp_ref[...], _DN,
                        preferred_element_type=jnp.float32)
    acc = fst[:, 0:1] * y[:, :D]
    for j in range(1, DE):
        acc += fst[:, j : j + 1] * y[:, j * D : (j + 1) * D]
    out_ref[...] = acc * scale


def _make_messages(ecount):
    return pl.pallas_call(
        _msg_body,
        grid=(ecount // _BE,),
        in_specs=[
            pl.BlockSpec((_BE, D), lambda i: (i, 0)),
            pl.BlockSpec((2 * DE, _BE), lambda i: (0, i)),
            pl.BlockSpec((D, DE * D), lambda i: (0, 0)),
            pl.BlockSpec((DE, D), lambda i: (0, 0)),
            pl.BlockSpec((1, D), lambda i: (0, 0)),
        ],
        out_specs=pl.BlockSpec((_BE, D), lambda i: (i, 0)),
        out_shape=jax.ShapeDtypeStruct((ecount, D), jnp.float32),
    )


_tc_messages1 = _make_messages(_E1)
_tc_messages2 = _make_messages(_E2)


_BN = 1000


def _fin_body(x_ref, a0_ref, a1_ref, wl_ref, out_ref):
    xa = x_ref[...] + a0_ref[0] + a1_ref[0]
    out_ref[...] = lax.dot_general(xa, wl_ref[...], _DN, preferred_element_type=jnp.float32)


def _fin_body4(x_ref, a0_ref, a1_ref, a2_ref, a3_ref, wl_ref, out_ref):
    xa = (x_ref[...] + a0_ref[0] + a1_ref[0]) + (a2_ref[0] + a3_ref[0])
    out_ref[...] = lax.dot_general(xa, wl_ref[...], _DN, preferred_element_type=jnp.float32)


_AGG_SPECS = [
    pl.BlockSpec((1, _BN, D), lambda i: (0, i, 0)),
    pl.BlockSpec((1, _BN, D), lambda i: (1, i, 0)),
]

_tc_final = pl.pallas_call(
    _fin_body4,
    grid=(N // _BN,),
    in_specs=[
        pl.BlockSpec((_BN, D), lambda i: (i, 0)),
        _AGG_SPECS[0],
        _AGG_SPECS[1],
        _AGG_SPECS[0],
        _AGG_SPECS[1],
        pl.BlockSpec((D, D), lambda i: (0, 0)),
    ],
    out_specs=pl.BlockSpec((_BN, D), lambda i: (i, 0)),
    out_shape=jax.ShapeDtypeStruct((N, D), jnp.float32),
)


def kernel(x, edge_index, edge_feat, edge_scalars, W_tp, W_radial, b_radial, W_lin):
    src = edge_index[0]
    dst = edge_index[1]
    src1 = src[:_E1].reshape(_NW, _NCH1, _CH)
    src2 = src[_E1:].reshape(_NW, _NCH2, _CH)
    dst1 = dst[:_E1].reshape(_NW, _NCH1, _CH)
    dst2 = dst[_E1:].reshape(_NW, _NCH2, _CH)
    fs = jnp.concatenate([edge_feat.T, edge_scalars.T], axis=0)
    wtp_flat = (W_tp.reshape(D, DE * D) * _INV_NORM).astype(jnp.bfloat16)
    br = b_radial.reshape(1, D)

    xs1 = _sc_gather1(x, src1)
    msg1 = _tc_messages1(xs1, fs[:, :_E1], wtp_flat, W_radial, br)
    xs2 = _sc_gather2(x, src2)
    msg2 = _tc_messages2(xs2, fs[:, _E1:], wtp_flat, W_radial, br)
    agg1 = _sc_scatter1(msg1, dst1)
    agg2 = _sc_scatter2(msg2, dst2)
    out = _tc_final(x, agg1, agg1, agg2, agg2, W_lin / math.sqrt(float(D)))
    return out


# speedup vs baseline: 3.9977x; 1.0286x over previous
import functools
import math

import jax
import jax.numpy as jnp
from jax import lax
from jax.experimental import pallas as pl
from jax.experimental.pallas import tpu as pltpu
from jax.experimental.pallas import tpu_sc as plsc

N = 10000
E = 160000
D = 128
DE = 4

_E1 = 83200
_E2 = 76800

_NC = 2
_NS = 16
_NW = _NC * _NS
_CH = 40
_NB = 5
_NCH1 = _E1 // _NW // _CH
_NCH2 = _E2 // _NW // _CH

_NP = 10240
_RPT = _NP // _NS
_ZB = 32
_NZB = _RPT // _ZB

_MESH = plsc.VectorSubcoreMesh(core_axis_name="c", subcore_axis_name="s")


def _make_gather(ecount, nch):
    ng = nch // _NB
    pw = ecount // _NW

    @functools.partial(
        pl.kernel,
        out_type=jax.ShapeDtypeStruct((ecount, D), jnp.float32),
        mesh=_MESH,
        scratch_types=[
            pltpu.VMEM((nch, _CH), jnp.int32),
            [pltpu.VMEM((_CH, D), jnp.float32) for _ in range(_NB)],
            pltpu.SemaphoreType.DMA((_NB,)),
            pltpu.SemaphoreType.DMA((_NB,)),
        ],
    )
    def _g(x_hbm, src_hbm, xs_hbm, idx_v, rows_v, gsem, ssem):
        wid = lax.axis_index("s") * _NC + lax.axis_index("c")
        base = wid * pw
        pltpu.sync_copy(src_hbm.at[wid], idx_v)

        def _gather(g, b):
            pltpu.async_copy(x_hbm.at[idx_v.at[g]], rows_v[b], gsem.at[b])

        def _store(g, b):
            off = pl.multiple_of(base + g * _CH, 8)
            pltpu.async_copy(rows_v[b], xs_hbm.at[pl.ds(off, _CH)], ssem.at[b])

        def _wait_gather(b):
            pltpu.make_async_copy(x_hbm.at[idx_v.at[0]], rows_v[b], gsem.at[b]).wait()

        def _wait_store(b):
            pltpu.make_async_copy(rows_v[b], xs_hbm.at[pl.ds(base, _CH)], ssem.at[b]).wait()

        for b in range(_NB):
            _gather(b, b)

        def body(k, carry):
            t = k * _NB
            for b in range(_NB):
                _wait_gather(b)
                _store(t + b, b)
            for b in range(_NB):
                _wait_store(b)

                @pl.when(k < ng - 1)
                def _():
                    _gather(t + _NB + b, b)
            return carry

        lax.fori_loop(0, ng, body, 0)

    return _g


_sc_gather1 = _make_gather(_E1, _NCH1)
_sc_gather2 = _make_gather(_E2, _NCH2)


def _make_scatter(ecount, nch):
    ng = nch // _NB
    pw = ecount // _NW

    @functools.partial(
        pl.kernel,
        out_type=jax.ShapeDtypeStruct((_NC, _NP, D), jnp.float32),
        mesh=_MESH,
        scratch_types=[
            pltpu.VMEM((nch, _CH), jnp.int32),
            [pltpu.VMEM((_CH, D), jnp.float32) for _ in range(_NB)],
            pltpu.VMEM((_ZB, D), jnp.float32),
            pltpu.VMEM_SHARED((_NP, D), jnp.float32),
            pltpu.SemaphoreType.DMA((_NB,)),
            pltpu.SemaphoreType.DMA((_NB,)),
        ],
    )
    def _sc(msg_hbm, dst_hbm, agg_hbm, idx_v, rows_v, znc_v, acc_sh, lsem, asem):
        c = lax.axis_index("c")
        s = lax.axis_index("s")
        wid = s * _NC + c
        base = wid * pw

        zvec = jnp.zeros((16,), jnp.float32)

        def zbody(i, carry):
            for cc in range(D // 16):
                znc_v[i, pl.ds(cc * 16, 16)] = zvec
            return carry

        lax.fori_loop(0, _ZB, zbody, 0)
        for b in range(_NZB):
            r0 = s * _RPT + b * _ZB
            pltpu.sync_copy(znc_v, acc_sh.at[pl.ds(r0, _ZB)])
        plsc.subcore_barrier()

        pltpu.sync_copy(dst_hbm.at[wid], idx_v)

        def _load(g, b):
            off = pl.multiple_of(base + g * _CH, 8)
            pltpu.async_copy(msg_hbm.at[pl.ds(off, _CH)], rows_v[b], lsem.at[b])

        def _scat(g, b):
            pltpu.async_copy(rows_v[b], acc_sh.at[idx_v.at[g]], asem.at[b], add=True)

        def _wait_load(b):
            pltpu.make_async_copy(msg_hbm.at[pl.ds(base, _CH)], rows_v[b], lsem.at[b]).wait()

        def _wait_scat(b):
            pltpu.make_async_copy(rows_v[b], acc_sh.at[idx_v.at[0]], asem.at[b]).wait()

        for b in range(_NB):
            _load(b, b)

        def body(k, carry):
            t = k * _NB
            for b in range(_NB):
                _wait_load(b)
                _scat(t + b, b)
            for b in range(_NB):
                _wait_scat(b)

                @pl.when(k < ng - 1)
                def _():
                    _load(t + _NB + b, b)
            return carry

        lax.fori_loop(0, ng, body, 0)
        plsc.subcore_barrier()

        for b in range(_NZB):
            r0 = s * _RPT + b * _ZB
            pltpu.sync_copy(acc_sh.at[pl.ds(r0, _ZB)], znc_v)
            pltpu.sync_copy(znc_v, agg_hbm.at[c, pl.ds(r0, _ZB)])

    return _sc


_sc_scatter1 = _make_scatter(_E1, _NCH1)
_sc_scatter2 = _make_scatter(_E2, _NCH2)


_BE = 1280
_INV_NORM = 1.0 / math.sqrt(float(D * DE))
_DN = (((1,), (0,)), ((), ()))


def _msg_body(xs_ref, fs_ref, wtp_ref, sel_ref, br_ref, out_ref):
    xs = xs_ref[...]
    fsb = fs_ref[...].astype(jnp.bfloat16)
    g = lax.dot_general(fsb, sel_ref[...], (((0,), (0,)), ((), ())),
                        preferred_element_type=jnp.float32)
    scale = 0.5 * jnp.tanh(0.5 * (g[:, DE * D :] + br_ref[...])) + 0.5
    y = lax.dot_general(xs.astype(jnp.bfloat16), wtp_ref[...], _DN,
                        preferred_element_type=jnp.float32)
    acc = g[:, :D] * y[:, :D]
    for j in range(1, DE):
        acc += g[:, j * D : (j + 1) * D] * y[:, j * D : (j + 1) * D]
    out_ref[...] = acc * scale


def _make_messages(ecount):
    return pl.pallas_call(
        _msg_body,
        grid=(ecount // _BE,),
        in_specs=[
            pl.BlockSpec((_BE, D), lambda i: (i, 0)),
            pl.BlockSpec((2 * DE, _BE), lambda i: (0, i)),
            pl.BlockSpec((D, DE * D), lambda i: (0, 0)),
            pl.BlockSpec((2 * DE, DE * D + D), lambda i: (0, 0)),
            pl.BlockSpec((1, D), lambda i: (0, 0)),
        ],
        out_specs=pl.BlockSpec((_BE, D), lambda i: (i, 0)),
        out_shape=jax.ShapeDtypeStruct((ecount, D), jnp.float32),
    )


_tc_messages1 = _make_messages(_E1)
_tc_messages2 = _make_messages(_E2)


_BN = 1000


def _fin_body(x_ref, a0_ref, a1_ref, wl_ref, out_ref):
    xa = x_ref[...] + a0_ref[0] + a1_ref[0]
    out_ref[...] = lax.dot_general(xa, wl_ref[...], _DN, preferred_element_type=jnp.float32)


def _fin_body4(x_ref, a0_ref, a1_ref, a2_ref, a3_ref, wl_ref, out_ref):
    xa = (x_ref[...] + a0_ref[0] + a1_ref[0]) + (a2_ref[0] + a3_ref[0])
    out_ref[...] = lax.dot_general(xa, wl_ref[...], _DN, preferred_element_type=jnp.float32)


_AGG_SPECS = [
    pl.BlockSpec((1, _BN, D), lambda i: (0, i, 0)),
    pl.BlockSpec((1, _BN, D), lambda i: (1, i, 0)),
]

_tc_final = pl.pallas_call(
    _fin_body4,
    grid=(N // _BN,),
    in_specs=[
        pl.BlockSpec((_BN, D), lambda i: (i, 0)),
        _AGG_SPECS[0],
        _AGG_SPECS[1],
        _AGG_SPECS[0],
        _AGG_SPECS[1],
        pl.BlockSpec((D, D), lambda i: (0, 0)),
    ],
    out_specs=pl.BlockSpec((_BN, D), lambda i: (i, 0)),
    out_shape=jax.ShapeDtypeStruct((N, D), jnp.float32),
)


def kernel(x, edge_index, edge_feat, edge_scalars, W_tp, W_radial, b_radial, W_lin):
    src = edge_index[0]
    dst = edge_index[1]
    src1 = src[:_E1].reshape(_NW, _NCH1, _CH)
    src2 = src[_E1:].reshape(_NW, _NCH2, _CH)
    dst1 = dst[:_E1].reshape(_NW, _NCH1, _CH)
    dst2 = dst[_E1:].reshape(_NW, _NCH2, _CH)
    fs = jnp.concatenate([edge_feat.T, edge_scalars.T], axis=0)
    wtp_flat = (W_tp.reshape(D, DE * D) * _INV_NORM).astype(jnp.bfloat16)
    br = b_radial.reshape(1, D)
    eye = jnp.eye(DE, dtype=jnp.float32)
    sel = jnp.concatenate(
        [jnp.kron(eye, jnp.ones((1, D), jnp.float32)),
         jnp.zeros((DE, D), jnp.float32)], axis=1)
    sel = jnp.concatenate(
        [sel, jnp.concatenate([jnp.zeros((DE, DE * D), jnp.float32), W_radial],
                              axis=1)], axis=0)
    sel = sel.astype(jnp.bfloat16)

    xs1 = _sc_gather1(x, src1)
    msg1 = _tc_messages1(xs1, fs[:, :_E1], wtp_flat, sel, br)
    xs2 = _sc_gather2(x, src2)
    msg2 = _tc_messages2(xs2, fs[:, _E1:], wtp_flat, sel, br)
    agg1 = _sc_scatter1(msg1, dst1)
    agg2 = _sc_scatter2(msg2, dst2)
    out = _tc_final(x, agg1, agg1, agg2, agg2, W_lin / math.sqrt(float(D)))
    return out


# speedup vs baseline: 4.0347x; 1.0093x over previous
import functools
import math

import jax
import jax.numpy as jnp
from jax import lax
from jax.experimental import pallas as pl
from jax.experimental.pallas import tpu as pltpu
from jax.experimental.pallas import tpu_sc as plsc

N = 10000
E = 160000
D = 128
DE = 4

_CHUNKS = (25600, 44800, 44800, 44800)
_OFFS = (0, 25600, 70400, 115200)

_NC = 2
_NS = 16
_NW = _NC * _NS
_CH = 40
_NB = 5

_NP = 10240
_RPT = _NP // _NS
_ZB = 32
_NZB = _RPT // _ZB

_MESH = plsc.VectorSubcoreMesh(core_axis_name="c", subcore_axis_name="s")


def _make_gather(ecount, nch):
    ng = nch // _NB
    pw = ecount // _NW

    @functools.partial(
        pl.kernel,
        out_type=jax.ShapeDtypeStruct((ecount, D), jnp.float32),
        mesh=_MESH,
        scratch_types=[
            pltpu.VMEM((nch, _CH), jnp.int32),
            [pltpu.VMEM((_CH, D), jnp.float32) for _ in range(_NB)],
            pltpu.SemaphoreType.DMA((_NB,)),
            pltpu.SemaphoreType.DMA((_NB,)),
        ],
    )
    def _g(x_hbm, src_hbm, xs_hbm, idx_v, rows_v, gsem, ssem):
        wid = lax.axis_index("s") * _NC + lax.axis_index("c")
        base = wid * pw
        pltpu.sync_copy(src_hbm.at[wid], idx_v)

        def _gather(g, b):
            pltpu.async_copy(x_hbm.at[idx_v.at[g]], rows_v[b], gsem.at[b])

        def _store(g, b):
            off = pl.multiple_of(base + g * _CH, 8)
            pltpu.async_copy(rows_v[b], xs_hbm.at[pl.ds(off, _CH)], ssem.at[b])

        def _wait_gather(b):
            pltpu.make_async_copy(x_hbm.at[idx_v.at[0]], rows_v[b], gsem.at[b]).wait()

        def _wait_store(b):
            pltpu.make_async_copy(rows_v[b], xs_hbm.at[pl.ds(base, _CH)], ssem.at[b]).wait()

        for b in range(_NB):
            _gather(b, b)

        def body(k, carry):
            t = k * _NB
            for b in range(_NB):
                _wait_gather(b)
                _store(t + b, b)
            for b in range(_NB):
                _wait_store(b)

                @pl.when(k < ng - 1)
                def _():
                    _gather(t + _NB + b, b)
            return carry

        lax.fori_loop(0, ng, body, 0)

    return _g


_gathers = [_make_gather(c, c // _NW // _CH) for c in _CHUNKS]


def _make_scatter(counts):
    n = len(counts)
    nchs = [cnt // _NW // _CH for cnt in counts]

    @functools.partial(
        pl.kernel,
        out_type=jax.ShapeDtypeStruct((_NC, _NP, D), jnp.float32),
        mesh=_MESH,
        scratch_types=[
            [pltpu.VMEM((nch, _CH), jnp.int32) for nch in nchs],
            [pltpu.VMEM((_CH, D), jnp.float32) for _ in range(_NB)],
            pltpu.VMEM((_ZB, D), jnp.float32),
            pltpu.VMEM_SHARED((_NP, D), jnp.float32),
            pltpu.SemaphoreType.DMA((_NB,)),
            pltpu.SemaphoreType.DMA((_NB,)),
        ],
    )
    def _sc(*refs):
        msgs = refs[0 : 2 * n : 2]
        dsts = refs[1 : 2 * n : 2]
        agg_hbm = refs[2 * n]
        idxs, rows_v, znc_v, acc_sh, lsem, asem = refs[2 * n + 1 :]
        c = lax.axis_index("c")
        s = lax.axis_index("s")
        wid = s * _NC + c

        zvec = jnp.zeros((16,), jnp.float32)

        def zbody(i, carry):
            for cc in range(D // 16):
                znc_v[i, pl.ds(cc * 16, 16)] = zvec
            return carry

        lax.fori_loop(0, _ZB, zbody, 0)
        for b in range(_NZB):
            r0 = s * _RPT + b * _ZB
            pltpu.sync_copy(znc_v, acc_sh.at[pl.ds(r0, _ZB)])
        plsc.subcore_barrier()

        def _run(msg_hbm, idx_v, nch):
            ng = nch // _NB
            base = wid * nch * _CH

            def _load(g, b):
                off = pl.multiple_of(base + g * _CH, 8)
                pltpu.async_copy(msg_hbm.at[pl.ds(off, _CH)], rows_v[b], lsem.at[b])

            def _scat(g, b):
                pltpu.async_copy(rows_v[b], acc_sh.at[idx_v.at[g]], asem.at[b], add=True)

            def _wait_load(b):
                pltpu.make_async_copy(msg_hbm.at[pl.ds(base, _CH)], rows_v[b], lsem.at[b]).wait()

            def _wait_scat(b):
                pltpu.make_async_copy(rows_v[b], acc_sh.at[idx_v.at[0]], asem.at[b]).wait()

            for b in range(_NB):
                _load(b, b)

            def body(k, carry):
                t = k * _NB
                for b in range(_NB):
                    _wait_load(b)
                    _scat(t + b, b)
                for b in range(_NB):
                    _wait_scat(b)

                    @pl.when(k < ng - 1)
                    def _():
                        _load(t + _NB + b, b)
                return carry

            lax.fori_loop(0, ng, body, 0)

        for i in range(n):
            pltpu.sync_copy(dsts[i].at[wid], idxs[i])
            _run(msgs[i], idxs[i], nchs[i])
        plsc.subcore_barrier()

        for b in range(_NZB):
            r0 = s * _RPT + b * _ZB
            pltpu.sync_copy(acc_sh.at[pl.ds(r0, _ZB)], znc_v)
            pltpu.sync_copy(znc_v, agg_hbm.at[c, pl.ds(r0, _ZB)])

    return _sc


_sc_scatter_a = _make_scatter(_CHUNKS[:3])
_sc_scatter_b = _make_scatter(_CHUNKS[3:])


_BE = 1280
_INV_NORM = 1.0 / math.sqrt(float(D * DE))
_DN = (((1,), (0,)), ((), ()))


def _msg_body(xs_ref, fs_ref, wtp_ref, sel_ref, br_ref, out_ref):
    xs = xs_ref[...]
    fsb = fs_ref[...].astype(jnp.bfloat16)
    g = lax.dot_general(fsb, sel_ref[...], (((0,), (0,)), ((), ())),
                        preferred_element_type=jnp.float32)
    scale = 0.5 * jnp.tanh(0.5 * (g[:, DE * D :] + br_ref[...])) + 0.5
    y = lax.dot_general(xs.astype(jnp.bfloat16), wtp_ref[...], _DN,
                        preferred_element_type=jnp.float32)
    acc = g[:, :D] * y[:, :D]
    for j in range(1, DE):
        acc += g[:, j * D : (j + 1) * D] * y[:, j * D : (j + 1) * D]
    out_ref[...] = acc * scale


def _make_messages(ecount):
    return pl.pallas_call(
        _msg_body,
        grid=(ecount // _BE,),
        in_specs=[
            pl.BlockSpec((_BE, D), lambda i: (i, 0)),
            pl.BlockSpec((2 * DE, _BE), lambda i: (0, i)),
            pl.BlockSpec((D, DE * D), lambda i: (0, 0)),
            pl.BlockSpec((2 * DE, DE * D + D), lambda i: (0, 0)),
            pl.BlockSpec((1, D), lambda i: (0, 0)),
        ],
        out_specs=pl.BlockSpec((_BE, D), lambda i: (i, 0)),
        out_shape=jax.ShapeDtypeStruct((ecount, D), jnp.float32),
    )


_messages = [_make_messages(c) for c in _CHUNKS]


_BN = 1000


def _fin_body(x_ref, a0_ref, a1_ref, wl_ref, out_ref):
    xa = x_ref[...] + a0_ref[0] + a1_ref[0]
    out_ref[...] = lax.dot_general(xa, wl_ref[...], _DN, preferred_element_type=jnp.float32)


def _fin_body4(x_ref, a0_ref, a1_ref, a2_ref, a3_ref, wl_ref, out_ref):
    xa = (x_ref[...] + a0_ref[0] + a1_ref[0]) + (a2_ref[0] + a3_ref[0])
    out_ref[...] = lax.dot_general(xa, wl_ref[...], _DN, preferred_element_type=jnp.float32)


_AGG_SPECS = [
    pl.BlockSpec((1, _BN, D), lambda i: (0, i, 0)),
    pl.BlockSpec((1, _BN, D), lambda i: (1, i, 0)),
]

_tc_final = pl.pallas_call(
    _fin_body4,
    grid=(N // _BN,),
    in_specs=[
        pl.BlockSpec((_BN, D), lambda i: (i, 0)),
        _AGG_SPECS[0],
        _AGG_SPECS[1],
        _AGG_SPECS[0],
        _AGG_SPECS[1],
        pl.BlockSpec((D, D), lambda i: (0, 0)),
    ],
    out_specs=pl.BlockSpec((_BN, D), lambda i: (i, 0)),
    out_shape=jax.ShapeDtypeStruct((N, D), jnp.float32),
)


def kernel(x, edge_index, edge_feat, edge_scalars, W_tp, W_radial, b_radial, W_lin):
    src = edge_index[0]
    dst = edge_index[1]
    srcs, dsts = [], []
    for o, cnt in zip(_OFFS, _CHUNKS):
        nch = cnt // _NW // _CH
        srcs.append(src[o : o + cnt].reshape(_NW, nch, _CH))
        dsts.append(dst[o : o + cnt].reshape(_NW, nch, _CH))
    fs = jnp.concatenate([edge_feat.T, edge_scalars.T], axis=0)
    wtp_flat = (W_tp.reshape(D, DE * D) * _INV_NORM).astype(jnp.bfloat16)
    br = b_radial.reshape(1, D)
    eye = jnp.eye(DE, dtype=jnp.float32)
    sel = jnp.concatenate(
        [jnp.kron(eye, jnp.ones((1, D), jnp.float32)),
         jnp.zeros((DE, D), jnp.float32)], axis=1)
    sel = jnp.concatenate(
        [sel, jnp.concatenate([jnp.zeros((DE, DE * D), jnp.float32), W_radial],
                              axis=1)], axis=0)
    sel = sel.astype(jnp.bfloat16)

    msg = []
    for i, (o, cnt) in enumerate(zip(_OFFS, _CHUNKS)):
        xs = _gathers[i](x, srcs[i])
        msg.append(_messages[i](xs, fs[:, o : o + cnt], wtp_flat, sel, br))
    agg_a = _sc_scatter_a(msg[0], dsts[0], msg[1], dsts[1], msg[2], dsts[2])
    agg_b = _sc_scatter_b(msg[3], dsts[3])
    out = _tc_final(x, agg_a, agg_a, agg_b, agg_b, W_lin / math.sqrt(float(D)))
    return out
